# baseline (speedup 1.0000x reference)
import functools

import numpy as np

import jax
import jax.numpy as jnp
from jax import lax
from jax.experimental import pallas as pl
from jax.experimental.pallas import tpu as pltpu
from jax.experimental.pallas import tpu_sc as plsc

D_IN = 224
D_H1 = 256
D_HALF = 128
D_H2 = 128
D_OUT = 65

X_BLK = 2000
R_BLK = 4096
CHUNK = 128

_HI_MASK = np.uint32(0xFFFF0000)


def _pack_bf16_pair(lo_f32, hi_f32):
    lo_bits = lax.bitcast_convert_type(lo_f32.astype(jnp.bfloat16).astype(jnp.float32), jnp.uint32)
    hi_bits = lax.bitcast_convert_type(hi_f32.astype(jnp.bfloat16).astype(jnp.float32), jnp.uint32)
    return (hi_bits & _HI_MASK) | (lo_bits >> 16)


def _unpack_bf16_pair(packed_u32):
    lo = lax.bitcast_convert_type(packed_u32 << 16, jnp.float32)
    hi = lax.bitcast_convert_type(packed_u32 & _HI_MASK, jnp.float32)
    return lo, hi


def _precompute_body(x1_ref, x2_ref, f1_ref, w_ref, rs_ref, b_ref, out_ref, *, d1, d2):
    w = w_ref[0] * rs_ref[...]
    acc = (jnp.dot(x1_ref[...], w[:d1], preferred_element_type=jnp.float32)
           + jnp.dot(x2_ref[...], w[d1:d1 + d2], preferred_element_type=jnp.float32)
           + jnp.dot(f1_ref[...], w[d1 + d2:], preferred_element_type=jnp.float32))
    acc = acc + b_ref[0]
    out_ref[...] = _pack_bf16_pair(acc[:, :D_HALF], acc[:, D_HALF:])


def _precompute(x1, x2, f1, w_st, rs, b_st, n_rows):
    grid_j = n_rows // X_BLK
    d1, d2 = x1.shape[1], x2.shape[1]
    return pl.pallas_call(
        functools.partial(_precompute_body, d1=d1, d2=d2),
        grid=(2, grid_j),
        in_specs=[
            pl.BlockSpec((X_BLK, d1), lambda i, j: (j, 0)),
            pl.BlockSpec((X_BLK, d2), lambda i, j: (j, 0)),
            pl.BlockSpec((X_BLK, D_IN - d1 - d2), lambda i, j: (j, 0)),
            pl.BlockSpec((1, D_IN, D_H1), lambda i, j: (i, 0, 0)),
            pl.BlockSpec((D_IN, 1), lambda i, j: (0, 0)),
            pl.BlockSpec((1, 1, D_H1), lambda i, j: (i, 0, 0)),
        ],
        out_specs=pl.BlockSpec((X_BLK, D_HALF), lambda i, j: (i * grid_j + j, 0)),
        out_shape=jax.ShapeDtypeStruct((2 * n_rows, D_HALF), jnp.uint32),
    )(x1, x2, f1, w_st, rs, b_st)


NBUF = 4


def _make_sc_gather(e2):
    info = plsc.get_sparse_core_info()
    nc, ns = info.num_cores, info.num_subcores
    nw = nc * ns
    per_w = e2 // nw
    n_chunks = per_w // CHUNK
    assert n_chunks % 4 == 0 and n_chunks >= 8
    mesh = plsc.VectorSubcoreMesh(core_axis_name="c", subcore_axis_name="s")

    @functools.partial(
        pl.kernel,
        mesh=mesh,
        out_type=jax.ShapeDtypeStruct((e2, D_HALF), jnp.uint32),
        scratch_types=[
            pltpu.VMEM((per_w,), jnp.int32),
            pltpu.VMEM((NBUF, CHUNK, D_HALF), jnp.uint32),
            [pltpu.SemaphoreType.DMA] * NBUF,
            [pltpu.SemaphoreType.DMA] * NBUF,
        ],
        compiler_params=pltpu.CompilerParams(use_tc_tiling_on_sc=True),
    )
    def sc_gather(p_hbm, idx_hbm, out_hbm, idx_all, rows_v, sg, ss):
        wid = lax.axis_index("s") * nc + lax.axis_index("c")
        w_base = wid * per_w
        pltpu.sync_copy(idx_hbm.at[pl.ds(w_base, per_w)], idx_all)

        def fire_gather(g, slot):
            idx_sl = idx_all.at[pl.ds(g * CHUNK, CHUNK)]
            pltpu.async_copy(p_hbm.at[idx_sl], rows_v.at[slot], sg[slot])

        def fire_store(g, slot):
            pltpu.async_copy(rows_v.at[slot],
                             out_hbm.at[pl.ds(w_base + g * CHUNK, CHUNK)], ss[slot])

        def wait_gather(slot):
            pltpu.make_async_copy(p_hbm.at[idx_all.at[pl.ds(0, CHUNK)]],
                                  rows_v.at[slot], sg[slot]).wait()

        def wait_store(slot):
            pltpu.make_async_copy(rows_v.at[slot],
                                  out_hbm.at[pl.ds(w_base, CHUNK)], ss[slot]).wait()

        fire_gather(0, 0)
        fire_gather(1, 1)
        for g in (0, 1):
            fire_gather(g + 2, g + 2)
            wait_gather(g)
            fire_store(g, g)

        def quad(h, _):
            for b4 in range(4):
                g = 2 + h * 4 + b4
                slot = (2 + b4) % 4
                wait_store((slot + 2) % 4)
                fire_gather(g + 2, (slot + 2) % 4)
                wait_gather(slot)
                fire_store(g, slot)
            return 0

        lax.fori_loop(0, (n_chunks - 4) // 4, quad, 0)

        for g in (n_chunks - 2, n_chunks - 1):
            slot = g % 4
            wait_store((slot + 2) % 4)
            wait_gather(slot)
            fire_store(g, slot)
        wait_store((n_chunks - 2) % 4)
        wait_store((n_chunks - 1) % 4)

    return sc_gather


def _bf16_split(x):
    hi = x.astype(jnp.bfloat16)
    lo = (x - hi.astype(jnp.float32)).astype(jnp.bfloat16)
    return hi, lo


def _dot_f32ish(x, w_hi, w_lo):
    x_bf = x.astype(jnp.bfloat16)
    return (jnp.dot(x_bf, w_hi, preferred_element_type=jnp.float32)
            + jnp.dot(x_bf, w_lo, preferred_element_type=jnp.float32))


def _mlp_body(sa_ref, sb_ref, w2lh_ref, w2ll_ref, w2hh_ref, w2hl_ref,
              b2_ref, w3h_ref, w3l_ref, b3_ref, out_ref):
    sal, sah = _unpack_bf16_pair(sa_ref[...])
    sbl, sbh = _unpack_bf16_pair(sb_ref[...])
    h1l = jnp.maximum(sal + sbl, 0.0)
    h1h = jnp.maximum(sah + sbh, 0.0)
    h2 = (_dot_f32ish(h1l, w2lh_ref[...], w2ll_ref[...])
          + _dot_f32ish(h1h, w2hh_ref[...], w2hl_ref[...]))
    h2 = jnp.maximum(h2 + b2_ref[...], 0.0)
    out_ref[...] = _dot_f32ish(h2, w3h_ref[...], w3l_ref[...]) + b3_ref[...]


def _mlp(s2, w2, b2, w3, b3, n_pairs, epad):
    grid = (n_pairs + R_BLK - 1) // R_BLK
    off = epad // R_BLK
    w2l_hi = w2[:D_HALF].astype(jnp.bfloat16)
    w2l_lo = (w2[:D_HALF] - w2l_hi.astype(jnp.float32)).astype(jnp.bfloat16)
    w2h_hi = w2[D_HALF:].astype(jnp.bfloat16)
    w2h_lo = (w2[D_HALF:] - w2h_hi.astype(jnp.float32)).astype(jnp.bfloat16)
    w3_hi = w3.astype(jnp.bfloat16)
    w3_lo = (w3 - w3_hi.astype(jnp.float32)).astype(jnp.bfloat16)
    wspec = pl.BlockSpec((D_HALF, D_H2), lambda g: (0, 0))
    return pl.pallas_call(
        _mlp_body,
        grid=(grid,),
        in_specs=[
            pl.BlockSpec((R_BLK, D_HALF), lambda g: (g, 0)),
            pl.BlockSpec((R_BLK, D_HALF), lambda g: (g + off, 0)),
            wspec, wspec, wspec, wspec,
            pl.BlockSpec((1, D_H2), lambda g: (0, 0)),
            pl.BlockSpec((D_H2, D_OUT), lambda g: (0, 0)),
            pl.BlockSpec((D_H2, D_OUT), lambda g: (0, 0)),
            pl.BlockSpec((1, D_OUT), lambda g: (0, 0)),
        ],
        out_specs=pl.BlockSpec((R_BLK, D_OUT), lambda g: (g, 0)),
        out_shape=jax.ShapeDtypeStruct((n_pairs, D_OUT), jnp.float32),
    )(s2, s2, w2l_hi, w2l_lo, w2h_hi, w2h_lo, b2, w3_hi, w3_lo, b3)


def kernel(x1_o, x2_o, idx, attt, features1, W1, b1, W2, b2, W3, b3):
    n = x1_o.shape[0]
    e = idx.shape[1]
    d1, d2 = x1_o.shape[1], x2_o.shape[1]

    rs = jnp.concatenate((
        jnp.full((d1, 1), 1.0, jnp.float32) * attt[0],
        jnp.full((d2, 1), 1.0, jnp.float32) * attt[1],
        jnp.ones((D_IN - d1 - d2, 1), jnp.float32),
    ), axis=0)
    w_st = jnp.stack((W1[:D_IN], W1[D_IN:]))
    b_st = jnp.stack((b1, jnp.zeros_like(b1)))[:, None, :]

    epad = ((e + R_BLK - 1) // R_BLK) * R_BLK
    pad = epad - e
    aa = jnp.pad(idx[0], (0, pad))
    bb = jnp.pad(idx[1], (0, pad)) + n
    idx_comb = jnp.concatenate((aa, bb))

    p = _precompute(x1_o, x2_o, features1, w_st, rs, b_st, n)

    s2 = _make_sc_gather(2 * epad)(p, idx_comb)

    return _mlp(s2, W2, b2[None, :], W3, b3[None, :], e, epad)


# speedup vs baseline: 1.0409x; 1.0409x over previous
import functools

import numpy as np

import jax
import jax.numpy as jnp
from jax import lax
from jax.experimental import pallas as pl
from jax.experimental.pallas import tpu as pltpu
from jax.experimental.pallas import tpu_sc as plsc

D_IN = 224
D_H1 = 256
D_HALF = 128
D_H2 = 128
D_OUT = 65

X_BLK = 2000
R_BLK = 8192
CHUNK = 128

_HI_MASK = np.uint32(0xFFFF0000)


def _pack_bf16_pair(lo_f32, hi_f32):
    lo_bits = lax.bitcast_convert_type(lo_f32.astype(jnp.bfloat16).astype(jnp.float32), jnp.uint32)
    hi_bits = lax.bitcast_convert_type(hi_f32.astype(jnp.bfloat16).astype(jnp.float32), jnp.uint32)
    return (hi_bits & _HI_MASK) | (lo_bits >> 16)


def _unpack_bf16_pair(packed_u32):
    lo = lax.bitcast_convert_type(packed_u32 << 16, jnp.float32)
    hi = lax.bitcast_convert_type(packed_u32 & _HI_MASK, jnp.float32)
    return lo, hi


def _precompute_body(x1_ref, x2_ref, f1_ref, w_ref, rs_ref, b_ref, out_ref, *, d1, d2):
    w = w_ref[0] * rs_ref[...]
    acc = (jnp.dot(x1_ref[...], w[:d1], preferred_element_type=jnp.float32)
           + jnp.dot(x2_ref[...], w[d1:d1 + d2], preferred_element_type=jnp.float32)
           + jnp.dot(f1_ref[...], w[d1 + d2:], preferred_element_type=jnp.float32))
    acc = acc + b_ref[0]
    out_ref[...] = _pack_bf16_pair(acc[:, :D_HALF], acc[:, D_HALF:])


def _precompute(x1, x2, f1, w_st, rs, b_st, n_rows):
    grid_j = n_rows // X_BLK
    d1, d2 = x1.shape[1], x2.shape[1]
    return pl.pallas_call(
        functools.partial(_precompute_body, d1=d1, d2=d2),
        grid=(2, grid_j),
        in_specs=[
            pl.BlockSpec((X_BLK, d1), lambda i, j: (j, 0)),
            pl.BlockSpec((X_BLK, d2), lambda i, j: (j, 0)),
            pl.BlockSpec((X_BLK, D_IN - d1 - d2), lambda i, j: (j, 0)),
            pl.BlockSpec((1, D_IN, D_H1), lambda i, j: (i, 0, 0)),
            pl.BlockSpec((D_IN, 1), lambda i, j: (0, 0)),
            pl.BlockSpec((1, 1, D_H1), lambda i, j: (i, 0, 0)),
        ],
        out_specs=pl.BlockSpec((X_BLK, D_HALF), lambda i, j: (i * grid_j + j, 0)),
        out_shape=jax.ShapeDtypeStruct((2 * n_rows, D_HALF), jnp.uint32),
    )(x1, x2, f1, w_st, rs, b_st)


NBUF = 4


def _make_sc_gather(e2):
    info = plsc.get_sparse_core_info()
    nc, ns = info.num_cores, info.num_subcores
    nw = nc * ns
    per_w = e2 // nw
    n_chunks = per_w // CHUNK
    assert n_chunks % 4 == 0 and n_chunks >= 8
    mesh = plsc.VectorSubcoreMesh(core_axis_name="c", subcore_axis_name="s")

    @functools.partial(
        pl.kernel,
        mesh=mesh,
        out_type=jax.ShapeDtypeStruct((e2, D_HALF), jnp.uint32),
        scratch_types=[
            pltpu.VMEM((per_w,), jnp.int32),
            pltpu.VMEM((NBUF, CHUNK, D_HALF), jnp.uint32),
            [pltpu.SemaphoreType.DMA] * NBUF,
            [pltpu.SemaphoreType.DMA] * NBUF,
        ],
        compiler_params=pltpu.CompilerParams(use_tc_tiling_on_sc=True),
    )
    def sc_gather(p_hbm, idx_hbm, out_hbm, idx_all, rows_v, sg, ss):
        wid = lax.axis_index("s") * nc + lax.axis_index("c")
        w_base = wid * per_w
        pltpu.sync_copy(idx_hbm.at[pl.ds(w_base, per_w)], idx_all)

        def fire_gather(g, slot):
            idx_sl = idx_all.at[pl.ds(g * CHUNK, CHUNK)]
            pltpu.async_copy(p_hbm.at[idx_sl], rows_v.at[slot], sg[slot])

        def fire_store(g, slot):
            pltpu.async_copy(rows_v.at[slot],
                             out_hbm.at[pl.ds(w_base + g * CHUNK, CHUNK)], ss[slot])

        def wait_gather(slot):
            pltpu.make_async_copy(p_hbm.at[idx_all.at[pl.ds(0, CHUNK)]],
                                  rows_v.at[slot], sg[slot]).wait()

        def wait_store(slot):
            pltpu.make_async_copy(rows_v.at[slot],
                                  out_hbm.at[pl.ds(w_base, CHUNK)], ss[slot]).wait()

        fire_gather(0, 0)
        fire_gather(1, 1)
        for g in (0, 1):
            fire_gather(g + 2, g + 2)
            wait_gather(g)
            fire_store(g, g)

        def quad(h, _):
            for b4 in range(4):
                g = 2 + h * 4 + b4
                slot = (2 + b4) % 4
                wait_store((slot + 2) % 4)
                fire_gather(g + 2, (slot + 2) % 4)
                wait_gather(slot)
                fire_store(g, slot)
            return 0

        lax.fori_loop(0, (n_chunks - 4) // 4, quad, 0)

        for g in (n_chunks - 2, n_chunks - 1):
            slot = g % 4
            wait_store((slot + 2) % 4)
            wait_gather(slot)
            fire_store(g, slot)
        wait_store((n_chunks - 2) % 4)
        wait_store((n_chunks - 1) % 4)

    return sc_gather


def _bf16_split(x):
    hi = x.astype(jnp.bfloat16)
    lo = (x - hi.astype(jnp.float32)).astype(jnp.bfloat16)
    return hi, lo


def _dot_f32ish(x, w_hi, w_lo):
    x_bf = x.astype(jnp.bfloat16)
    return (jnp.dot(x_bf, w_hi, preferred_element_type=jnp.float32)
            + jnp.dot(x_bf, w_lo, preferred_element_type=jnp.float32))


def _mlp_body(s_ref, w2lh_ref, w2ll_ref, w2hh_ref, w2hl_ref,
              b2_ref, w3h_ref, w3l_ref, b3_ref, out_ref):
    sal, sah = _unpack_bf16_pair(s_ref[0])
    sbl, sbh = _unpack_bf16_pair(s_ref[1])
    h1l = jnp.maximum(sal + sbl, 0.0)
    h1h = jnp.maximum(sah + sbh, 0.0)
    h2 = (_dot_f32ish(h1l, w2lh_ref[...], w2ll_ref[...])
          + _dot_f32ish(h1h, w2hh_ref[...], w2hl_ref[...]))
    h2 = jnp.maximum(h2 + b2_ref[...], 0.0)
    out_ref[...] = _dot_f32ish(h2, w3h_ref[...], w3l_ref[...]) + b3_ref[...]


def _mlp(s2, w2, b2, w3, b3, n_pairs, epad):
    grid = (n_pairs + R_BLK - 1) // R_BLK
    s2_3d = s2.reshape(2, epad, D_HALF)
    w2l_hi = w2[:D_HALF].astype(jnp.bfloat16)
    w2l_lo = (w2[:D_HALF] - w2l_hi.astype(jnp.float32)).astype(jnp.bfloat16)
    w2h_hi = w2[D_HALF:].astype(jnp.bfloat16)
    w2h_lo = (w2[D_HALF:] - w2h_hi.astype(jnp.float32)).astype(jnp.bfloat16)
    w3_hi = w3.astype(jnp.bfloat16)
    w3_lo = (w3 - w3_hi.astype(jnp.float32)).astype(jnp.bfloat16)
    wspec = pl.BlockSpec((D_HALF, D_H2), lambda g: (0, 0))
    return pl.pallas_call(
        _mlp_body,
        grid=(grid,),
        in_specs=[
            pl.BlockSpec((2, R_BLK, D_HALF), lambda g: (0, g, 0)),
            wspec, wspec, wspec, wspec,
            pl.BlockSpec((1, D_H2), lambda g: (0, 0)),
            pl.BlockSpec((D_H2, D_OUT), lambda g: (0, 0)),
            pl.BlockSpec((D_H2, D_OUT), lambda g: (0, 0)),
            pl.BlockSpec((1, D_OUT), lambda g: (0, 0)),
        ],
        out_specs=pl.BlockSpec((R_BLK, D_OUT), lambda g: (g, 0)),
        out_shape=jax.ShapeDtypeStruct((n_pairs, D_OUT), jnp.float32),
    )(s2_3d, w2l_hi, w2l_lo, w2h_hi, w2h_lo, b2, w3_hi, w3_lo, b3)


def kernel(x1_o, x2_o, idx, attt, features1, W1, b1, W2, b2, W3, b3):
    n = x1_o.shape[0]
    e = idx.shape[1]
    d1, d2 = x1_o.shape[1], x2_o.shape[1]

    rs = jnp.concatenate((
        jnp.full((d1, 1), 1.0, jnp.float32) * attt[0],
        jnp.full((d2, 1), 1.0, jnp.float32) * attt[1],
        jnp.ones((D_IN - d1 - d2, 1), jnp.float32),
    ), axis=0)
    w_st = jnp.stack((W1[:D_IN], W1[D_IN:]))
    b_st = jnp.stack((b1, jnp.zeros_like(b1)))[:, None, :]

    epad = ((e + R_BLK - 1) // R_BLK) * R_BLK
    pad = epad - e
    aa = jnp.pad(idx[0], (0, pad))
    bb = jnp.pad(idx[1], (0, pad)) + n
    idx_comb = jnp.concatenate((aa, bb))

    p = _precompute(x1_o, x2_o, features1, w_st, rs, b_st, n)

    s2 = _make_sc_gather(2 * epad)(p, idx_comb)

    return _mlp(s2, W2, b2[None, :], W3, b3[None, :], e, epad)


# speedup vs baseline: 1.3288x; 1.2767x over previous
import functools

import numpy as np

import jax
import jax.numpy as jnp
from jax import lax
from jax.experimental import pallas as pl
from jax.experimental.pallas import tpu as pltpu
from jax.experimental.pallas import tpu_sc as plsc

D_IN = 224
D_H1 = 256
D_HALF = 128
D_H2 = 128
D_OUT = 65

X_BLK = 2000
R_BLK = 8192
CHUNK = 128

_HI_MASK = np.uint32(0xFFFF0000)


def _pack_bf16_pair(lo_f32, hi_f32):
    lo_bits = lax.bitcast_convert_type(lo_f32.astype(jnp.bfloat16).astype(jnp.float32), jnp.uint32)
    hi_bits = lax.bitcast_convert_type(hi_f32.astype(jnp.bfloat16).astype(jnp.float32), jnp.uint32)
    return (hi_bits & _HI_MASK) | (lo_bits >> 16)


def _unpack_bf16_pair(packed_u32):
    lo = lax.bitcast_convert_type(packed_u32 << 16, jnp.float32)
    hi = lax.bitcast_convert_type(packed_u32 & _HI_MASK, jnp.float32)
    return lo, hi


def _precompute_body(x1_ref, x2_ref, f1_ref, w_ref, rs_ref, b_ref, out_ref, *, d1, d2):
    w = w_ref[0] * rs_ref[...]
    acc = (jnp.dot(x1_ref[...], w[:d1], preferred_element_type=jnp.float32)
           + jnp.dot(x2_ref[...], w[d1:d1 + d2], preferred_element_type=jnp.float32)
           + jnp.dot(f1_ref[...], w[d1 + d2:], preferred_element_type=jnp.float32))
    acc = acc + b_ref[0]
    out_ref[...] = _pack_bf16_pair(acc[:, :D_HALF], acc[:, D_HALF:])


def _precompute(x1, x2, f1, w_st, rs, b_st, n_rows):
    grid_j = n_rows // X_BLK
    d1, d2 = x1.shape[1], x2.shape[1]
    return pl.pallas_call(
        functools.partial(_precompute_body, d1=d1, d2=d2),
        grid=(2, grid_j),
        in_specs=[
            pl.BlockSpec((X_BLK, d1), lambda i, j: (j, 0)),
            pl.BlockSpec((X_BLK, d2), lambda i, j: (j, 0)),
            pl.BlockSpec((X_BLK, D_IN - d1 - d2), lambda i, j: (j, 0)),
            pl.BlockSpec((1, D_IN, D_H1), lambda i, j: (i, 0, 0)),
            pl.BlockSpec((D_IN, 1), lambda i, j: (0, 0)),
            pl.BlockSpec((1, 1, D_H1), lambda i, j: (i, 0, 0)),
        ],
        out_specs=pl.BlockSpec((X_BLK, D_HALF), lambda i, j: (i * grid_j + j, 0)),
        out_shape=jax.ShapeDtypeStruct((2 * n_rows, D_HALF), jnp.uint32),
    )(x1, x2, f1, w_st, rs, b_st)


NBUF = 4


def _make_sc_gather(e2):
    info = plsc.get_sparse_core_info()
    nc, ns = info.num_cores, info.num_subcores
    nw = nc * ns
    per_w = e2 // nw
    n_chunks = per_w // CHUNK
    assert n_chunks % 4 == 0 and n_chunks >= 8
    mesh = plsc.VectorSubcoreMesh(core_axis_name="c", subcore_axis_name="s")

    @functools.partial(
        pl.kernel,
        mesh=mesh,
        out_type=jax.ShapeDtypeStruct((e2, D_HALF), jnp.uint32),
        scratch_types=[
            pltpu.VMEM((per_w,), jnp.int32),
            pltpu.VMEM((NBUF, CHUNK, D_HALF), jnp.uint32),
            [pltpu.SemaphoreType.DMA] * NBUF,
            [pltpu.SemaphoreType.DMA] * NBUF,
        ],
        compiler_params=pltpu.CompilerParams(use_tc_tiling_on_sc=True),
    )
    def sc_gather(p_hbm, idx_hbm, out_hbm, idx_all, rows_v, sg, ss):
        wid = lax.axis_index("s") * nc + lax.axis_index("c")
        w_base = wid * per_w
        pltpu.sync_copy(idx_hbm.at[pl.ds(w_base, per_w)], idx_all)

        def fire_gather(g, slot):
            idx_sl = idx_all.at[pl.ds(g * CHUNK, CHUNK)]
            pltpu.async_copy(p_hbm.at[idx_sl], rows_v.at[slot], sg[slot])

        def fire_store(g, slot):
            pltpu.async_copy(rows_v.at[slot],
                             out_hbm.at[pl.ds(w_base + g * CHUNK, CHUNK)], ss[slot])

        def wait_gather(slot):
            pltpu.make_async_copy(p_hbm.at[idx_all.at[pl.ds(0, CHUNK)]],
                                  rows_v.at[slot], sg[slot]).wait()

        def wait_store(slot):
            pltpu.make_async_copy(rows_v.at[slot],
                                  out_hbm.at[pl.ds(w_base, CHUNK)], ss[slot]).wait()

        fire_gather(0, 0)
        fire_gather(1, 1)
        for g in (0, 1):
            fire_gather(g + 2, g + 2)
            wait_gather(g)
            fire_store(g, g)

        def quad(h, _):
            for b4 in range(4):
                g = 2 + h * 4 + b4
                slot = (2 + b4) % 4
                wait_store((slot + 2) % 4)
                fire_gather(g + 2, (slot + 2) % 4)
                wait_gather(slot)
                fire_store(g, slot)
            return 0

        lax.fori_loop(0, (n_chunks - 4) // 4, quad, 0)

        for g in (n_chunks - 2, n_chunks - 1):
            slot = g % 4
            wait_store((slot + 2) % 4)
            wait_gather(slot)
            fire_store(g, slot)
        wait_store((n_chunks - 2) % 4)
        wait_store((n_chunks - 1) % 4)

    return sc_gather


def _bf16_split(x):
    hi = x.astype(jnp.bfloat16)
    lo = (x - hi.astype(jnp.float32)).astype(jnp.bfloat16)
    return hi, lo


def _dot_f32ish(x, w_hi, w_lo):
    x_bf = x.astype(jnp.bfloat16)
    return (jnp.dot(x_bf, w_hi, preferred_element_type=jnp.float32)
            + jnp.dot(x_bf, w_lo, preferred_element_type=jnp.float32))


def _dot_t(w, x, w_dim, x_dim):
    return lax.dot_general(w, x, (((w_dim,), (x_dim,)), ((), ())),
                           preferred_element_type=jnp.float32)


def _mlp_body(s_ref, w2lh_ref, w2ll_ref, w2hh_ref, w2hl_ref,
              b2t_ref, w3h_ref, w3l_ref, b3t_ref, out_ref):
    sal, sah = _unpack_bf16_pair(s_ref[0])
    sbl, sbh = _unpack_bf16_pair(s_ref[1])
    h1l = jnp.maximum(sal + sbl, 0.0).astype(jnp.bfloat16)
    h1h = jnp.maximum(sah + sbh, 0.0).astype(jnp.bfloat16)
    h2t = (_dot_t(w2lh_ref[...], h1l, 0, 1) + _dot_t(w2ll_ref[...], h1l, 0, 1)
           + _dot_t(w2hh_ref[...], h1h, 0, 1) + _dot_t(w2hl_ref[...], h1h, 0, 1))
    h2t = jnp.maximum(h2t + b2t_ref[...], 0.0).astype(jnp.bfloat16)
    out_ref[...] = (_dot_t(w3h_ref[...], h2t, 0, 0) + _dot_t(w3l_ref[...], h2t, 0, 0)
                    + b3t_ref[...])


def _mlp(s2, w2, b2, w3, b3, n_pairs, epad):
    grid = (n_pairs + R_BLK - 1) // R_BLK
    s2_3d = s2.reshape(2, epad, D_HALF)
    w2l_hi = w2[:D_HALF].astype(jnp.bfloat16)
    w2l_lo = (w2[:D_HALF] - w2l_hi.astype(jnp.float32)).astype(jnp.bfloat16)
    w2h_hi = w2[D_HALF:].astype(jnp.bfloat16)
    w2h_lo = (w2[D_HALF:] - w2h_hi.astype(jnp.float32)).astype(jnp.bfloat16)
    w3_hi = w3.astype(jnp.bfloat16)
    w3_lo = (w3 - w3_hi.astype(jnp.float32)).astype(jnp.bfloat16)
    wspec = pl.BlockSpec((D_HALF, D_H2), lambda g: (0, 0))
    out_t = pl.pallas_call(
        _mlp_body,
        grid=(grid,),
        in_specs=[
            pl.BlockSpec((2, R_BLK, D_HALF), lambda g: (0, g, 0)),
            wspec, wspec, wspec, wspec,
            pl.BlockSpec((D_H2, 1), lambda g: (0, 0)),
            pl.BlockSpec((D_H2, D_OUT), lambda g: (0, 0)),
            pl.BlockSpec((D_H2, D_OUT), lambda g: (0, 0)),
            pl.BlockSpec((D_OUT, 1), lambda g: (0, 0)),
        ],
        out_specs=pl.BlockSpec((D_OUT, R_BLK), lambda g: (0, g)),
        out_shape=jax.ShapeDtypeStruct((D_OUT, n_pairs), jnp.float32),
    )(s2_3d, w2l_hi, w2l_lo, w2h_hi, w2h_lo, b2, w3_hi, w3_lo, b3)
    return out_t.T


def kernel(x1_o, x2_o, idx, attt, features1, W1, b1, W2, b2, W3, b3):
    n = x1_o.shape[0]
    e = idx.shape[1]
    d1, d2 = x1_o.shape[1], x2_o.shape[1]

    rs = jnp.concatenate((
        jnp.full((d1, 1), 1.0, jnp.float32) * attt[0],
        jnp.full((d2, 1), 1.0, jnp.float32) * attt[1],
        jnp.ones((D_IN - d1 - d2, 1), jnp.float32),
    ), axis=0)
    w_st = jnp.stack((W1[:D_IN], W1[D_IN:]))
    b_st = jnp.stack((b1, jnp.zeros_like(b1)))[:, None, :]

    epad = ((e + R_BLK - 1) // R_BLK) * R_BLK
    pad = epad - e
    aa = jnp.pad(idx[0], (0, pad))
    bb = jnp.pad(idx[1], (0, pad)) + n
    idx_comb = jnp.concatenate((aa, bb))

    p = _precompute(x1_o, x2_o, features1, w_st, rs, b_st, n)

    s2 = _make_sc_gather(2 * epad)(p, idx_comb)

    return _mlp(s2, W2, b2[:, None], W3, b3[:, None], e, epad)


# speedup vs baseline: 1.4057x; 1.0578x over previous
import functools

import numpy as np

import jax
import jax.numpy as jnp
from jax import lax
from jax.experimental import pallas as pl
from jax.experimental.pallas import tpu as pltpu
from jax.experimental.pallas import tpu_sc as plsc

D_IN = 224
D_H1 = 256
D_HALF = 128
D_H2 = 128
D_OUT = 65

X_BLK = 2000
R_BLK = 8192
CHUNK = 128

_HI_MASK = np.uint32(0xFFFF0000)


def _pack_bf16_pair(lo_f32, hi_f32):
    lo_bits = lax.bitcast_convert_type(lo_f32.astype(jnp.bfloat16).astype(jnp.float32), jnp.uint32)
    hi_bits = lax.bitcast_convert_type(hi_f32.astype(jnp.bfloat16).astype(jnp.float32), jnp.uint32)
    return (hi_bits & _HI_MASK) | (lo_bits >> 16)


def _unpack_bf16_pair(packed_u32):
    lo = lax.bitcast_convert_type(packed_u32 << 16, jnp.float32)
    hi = lax.bitcast_convert_type(packed_u32 & _HI_MASK, jnp.float32)
    return lo, hi


def _precompute_body(x1_ref, x2_ref, f1_ref, w_ref, rs_ref, b_ref, out_ref, *, d1, d2):
    w = w_ref[0] * rs_ref[...]
    acc = (jnp.dot(x1_ref[...], w[:d1], preferred_element_type=jnp.float32)
           + jnp.dot(x2_ref[...], w[d1:d1 + d2], preferred_element_type=jnp.float32)
           + jnp.dot(f1_ref[...], w[d1 + d2:], preferred_element_type=jnp.float32))
    acc = acc + b_ref[0]
    out_ref[...] = _pack_bf16_pair(acc[:, :D_HALF], acc[:, D_HALF:])


def _precompute(x1, x2, f1, w_st, rs, b_st, n_rows):
    grid_j = n_rows // X_BLK
    d1, d2 = x1.shape[1], x2.shape[1]
    return pl.pallas_call(
        functools.partial(_precompute_body, d1=d1, d2=d2),
        grid=(2, grid_j),
        in_specs=[
            pl.BlockSpec((X_BLK, d1), lambda i, j: (j, 0)),
            pl.BlockSpec((X_BLK, d2), lambda i, j: (j, 0)),
            pl.BlockSpec((X_BLK, D_IN - d1 - d2), lambda i, j: (j, 0)),
            pl.BlockSpec((1, D_IN, D_H1), lambda i, j: (i, 0, 0)),
            pl.BlockSpec((D_IN, 1), lambda i, j: (0, 0)),
            pl.BlockSpec((1, 1, D_H1), lambda i, j: (i, 0, 0)),
        ],
        out_specs=pl.BlockSpec((X_BLK, D_HALF), lambda i, j: (i * grid_j + j, 0)),
        out_shape=jax.ShapeDtypeStruct((2 * n_rows, D_HALF), jnp.uint32),
    )(x1, x2, f1, w_st, rs, b_st)


NBUF = 4


def _make_sc_gather(e2):
    info = plsc.get_sparse_core_info()
    nc, ns = info.num_cores, info.num_subcores
    nw = nc * ns
    per_w = e2 // nw
    n_chunks = per_w // CHUNK
    assert n_chunks % 4 == 0 and n_chunks >= 8
    mesh = plsc.VectorSubcoreMesh(core_axis_name="c", subcore_axis_name="s")

    @functools.partial(
        pl.kernel,
        mesh=mesh,
        out_type=jax.ShapeDtypeStruct((e2, D_HALF), jnp.uint32),
        scratch_types=[
            pltpu.VMEM((per_w,), jnp.int32),
            pltpu.VMEM((NBUF, CHUNK, D_HALF), jnp.uint32),
            [pltpu.SemaphoreType.DMA] * NBUF,
            [pltpu.SemaphoreType.DMA] * NBUF,
        ],
        compiler_params=pltpu.CompilerParams(use_tc_tiling_on_sc=True),
    )
    def sc_gather(p_hbm, idx_hbm, out_hbm, idx_all, rows_v, sg, ss):
        wid = lax.axis_index("s") * nc + lax.axis_index("c")
        w_base = wid * per_w
        pltpu.sync_copy(idx_hbm.at[pl.ds(w_base, per_w)], idx_all)

        def fire_gather(g, slot):
            idx_sl = idx_all.at[pl.ds(g * CHUNK, CHUNK)]
            pltpu.async_copy(p_hbm.at[idx_sl], rows_v.at[slot], sg[slot])

        def fire_store(g, slot):
            pltpu.async_copy(rows_v.at[slot],
                             out_hbm.at[pl.ds(w_base + g * CHUNK, CHUNK)], ss[slot])

        def wait_gather(slot):
            pltpu.make_async_copy(p_hbm.at[idx_all.at[pl.ds(0, CHUNK)]],
                                  rows_v.at[slot], sg[slot]).wait()

        def wait_store(slot):
            pltpu.make_async_copy(rows_v.at[slot],
                                  out_hbm.at[pl.ds(w_base, CHUNK)], ss[slot]).wait()

        fire_gather(0, 0)
        fire_gather(1, 1)
        for g in (0, 1):
            fire_gather(g + 2, g + 2)
            wait_gather(g)
            fire_store(g, g)

        def quad(h, _):
            for b4 in range(4):
                g = 2 + h * 4 + b4
                slot = (2 + b4) % 4
                wait_store((slot + 2) % 4)
                fire_gather(g + 2, (slot + 2) % 4)
                wait_gather(slot)
                fire_store(g, slot)
            return 0

        lax.fori_loop(0, (n_chunks - 4) // 4, quad, 0)

        for g in (n_chunks - 2, n_chunks - 1):
            slot = g % 4
            wait_store((slot + 2) % 4)
            wait_gather(slot)
            fire_store(g, slot)
        wait_store((n_chunks - 2) % 4)
        wait_store((n_chunks - 1) % 4)

    return sc_gather


def _bf16_split(x):
    hi = x.astype(jnp.bfloat16)
    lo = (x - hi.astype(jnp.float32)).astype(jnp.bfloat16)
    return hi, lo


def _dot_f32ish(x, w_hi, w_lo):
    x_bf = x.astype(jnp.bfloat16)
    return (jnp.dot(x_bf, w_hi, preferred_element_type=jnp.float32)
            + jnp.dot(x_bf, w_lo, preferred_element_type=jnp.float32))


def _dot_t(w, x, w_dim, x_dim):
    return lax.dot_general(w, x, (((w_dim,), (x_dim,)), ((), ())),
                           preferred_element_type=jnp.float32)


def _mlp_compute(s_ref, w2lh_ref, w2ll_ref, w2hh_ref, w2hl_ref,
                 b2t_ref, w3h_ref, w3l_ref, b3t_ref, out_ref):
    sal, sah = _unpack_bf16_pair(s_ref[0])
    sbl, sbh = _unpack_bf16_pair(s_ref[1])
    h1l = jnp.maximum(sal + sbl, 0.0).astype(jnp.bfloat16)
    h1h = jnp.maximum(sah + sbh, 0.0).astype(jnp.bfloat16)
    h2t = (_dot_t(w2lh_ref[...], h1l, 0, 1) + _dot_t(w2ll_ref[...], h1l, 0, 1)
           + _dot_t(w2hh_ref[...], h1h, 0, 1) + _dot_t(w2hl_ref[...], h1h, 0, 1))
    h2t = jnp.maximum(h2t + b2t_ref[...], 0.0).astype(jnp.bfloat16)
    out_ref[...] = (_dot_t(w3h_ref[...], h2t, 0, 0) + _dot_t(w3l_ref[...], h2t, 0, 0)
                    + b3t_ref[...])


def _mlp_body_first(*refs):
    _mlp_compute(*refs)


def _mlp_body_next(*refs):
    _mlp_compute(*refs[:9], refs[10])


def _mlp_chunk(s2, w2s, b2, w3s, b3, n_pairs, epad_k, n_blocks, col_base, prev):
    s2_3d = s2.reshape(2, epad_k, D_HALF)
    wspec = pl.BlockSpec((D_HALF, D_H2), lambda g: (0, 0))
    in_specs = [
        pl.BlockSpec((2, R_BLK, D_HALF), lambda g: (0, g, 0)),
        wspec, wspec, wspec, wspec,
        pl.BlockSpec((D_H2, 1), lambda g: (0, 0)),
        pl.BlockSpec((D_H2, D_OUT), lambda g: (0, 0)),
        pl.BlockSpec((D_H2, D_OUT), lambda g: (0, 0)),
        pl.BlockSpec((D_OUT, 1), lambda g: (0, 0)),
    ]
    args = [s2_3d, *w2s, b2, *w3s, b3]
    body = _mlp_body_first
    aliases = {}
    if prev is not None:
        in_specs.append(pl.BlockSpec(memory_space=pl.ANY))
        args.append(prev)
        body = _mlp_body_next
        aliases = {9: 0}
    return pl.pallas_call(
        body,
        grid=(n_blocks,),
        in_specs=in_specs,
        out_specs=pl.BlockSpec((D_OUT, R_BLK), lambda g: (0, g + col_base)),
        out_shape=jax.ShapeDtypeStruct((D_OUT, n_pairs), jnp.float32),
        input_output_aliases=aliases,
    )(*args)


def kernel(x1_o, x2_o, idx, attt, features1, W1, b1, W2, b2, W3, b3):
    n = x1_o.shape[0]
    e = idx.shape[1]
    d1, d2 = x1_o.shape[1], x2_o.shape[1]

    rs = jnp.concatenate((
        jnp.full((d1, 1), 1.0, jnp.float32) * attt[0],
        jnp.full((d2, 1), 1.0, jnp.float32) * attt[1],
        jnp.ones((D_IN - d1 - d2, 1), jnp.float32),
    ), axis=0)
    w_st = jnp.stack((W1[:D_IN], W1[D_IN:]))
    b_st = jnp.stack((b1, jnp.zeros_like(b1)))[:, None, :]

    p = _precompute(x1_o, x2_o, features1, w_st, rs, b_st, n)

    w2s = (
        W2[:D_HALF].astype(jnp.bfloat16),
        (W2[:D_HALF] - W2[:D_HALF].astype(jnp.bfloat16).astype(jnp.float32)).astype(jnp.bfloat16),
        W2[D_HALF:].astype(jnp.bfloat16),
        (W2[D_HALF:] - W2[D_HALF:].astype(jnp.bfloat16).astype(jnp.float32)).astype(jnp.bfloat16),
    )
    w3s = (
        W3.astype(jnp.bfloat16),
        (W3 - W3.astype(jnp.bfloat16).astype(jnp.float32)).astype(jnp.bfloat16),
    )
    b2t, b3t = b2[:, None], b3[:, None]

    e0 = (e // (2 * R_BLK)) * R_BLK
    chunks = [(0, e0), (e0, e - e0)] if 0 < e0 < e else [(0, e)]

    prev = None
    for start, sz in chunks:
        epad_k = ((sz + R_BLK - 1) // R_BLK) * R_BLK
        aa_k = jnp.pad(lax.dynamic_slice_in_dim(idx[0], start, sz), (0, epad_k - sz))
        bb_k = jnp.pad(lax.dynamic_slice_in_dim(idx[1], start, sz), (0, epad_k - sz)) + n
        idx_k = jnp.concatenate((aa_k, bb_k))

        s2_k = _make_sc_gather(2 * epad_k)(p, idx_k)

        prev = _mlp_chunk(s2_k, w2s, b2t, w3s, b3t, e, epad_k,
                          epad_k // R_BLK, start // R_BLK, prev)

    return prev.T


# speedup vs baseline: 1.7090x; 1.2158x over previous
import functools

import numpy as np

import jax
import jax.numpy as jnp
from jax import lax
from jax.experimental import pallas as pl
from jax.experimental.pallas import tpu as pltpu
from jax.experimental.pallas import tpu_sc as plsc

D_IN = 224
D_H1 = 256
D_HALF = 128
D_H2 = 128
D_OUT = 65

X_BLK = 2000
R_BLK = 8192
CHUNK = 128

_HI_MASK = np.uint32(0xFFFF0000)


def _pack_bf16_pair(lo_f32, hi_f32):
    lo_bits = lax.bitcast_convert_type(lo_f32.astype(jnp.bfloat16).astype(jnp.float32), jnp.uint32)
    hi_bits = lax.bitcast_convert_type(hi_f32.astype(jnp.bfloat16).astype(jnp.float32), jnp.uint32)
    return (hi_bits & _HI_MASK) | (lo_bits >> 16)


def _unpack_bf16_pair(packed_u32):
    lo = lax.bitcast_convert_type(packed_u32 << 16, jnp.float32)
    hi = lax.bitcast_convert_type(packed_u32 & _HI_MASK, jnp.float32)
    return lo, hi


def _precompute_body(x1_ref, x2_ref, f1_ref, w_ref, rs_ref, b_ref, out_ref, *, d1, d2):
    w = w_ref[0] * rs_ref[...]
    acc = (jnp.dot(x1_ref[...], w[:d1], preferred_element_type=jnp.float32)
           + jnp.dot(x2_ref[...], w[d1:d1 + d2], preferred_element_type=jnp.float32)
           + jnp.dot(f1_ref[...], w[d1 + d2:], preferred_element_type=jnp.float32))
    acc = acc + b_ref[0]
    out_ref[...] = _pack_bf16_pair(acc[:, :D_HALF], acc[:, D_HALF:])


def _precompute(x1, x2, f1, w_st, rs, b_st, n_rows):
    grid_j = n_rows // X_BLK
    d1, d2 = x1.shape[1], x2.shape[1]
    return pl.pallas_call(
        functools.partial(_precompute_body, d1=d1, d2=d2),
        grid=(2, grid_j),
        in_specs=[
            pl.BlockSpec((X_BLK, d1), lambda i, j: (j, 0)),
            pl.BlockSpec((X_BLK, d2), lambda i, j: (j, 0)),
            pl.BlockSpec((X_BLK, D_IN - d1 - d2), lambda i, j: (j, 0)),
            pl.BlockSpec((1, D_IN, D_H1), lambda i, j: (i, 0, 0)),
            pl.BlockSpec((D_IN, 1), lambda i, j: (0, 0)),
            pl.BlockSpec((1, 1, D_H1), lambda i, j: (i, 0, 0)),
        ],
        out_specs=pl.BlockSpec((X_BLK, D_HALF), lambda i, j: (i * grid_j + j, 0)),
        out_shape=jax.ShapeDtypeStruct((2 * n_rows, D_HALF), jnp.uint32),
    )(x1, x2, f1, w_st, rs, b_st)


NBUF = 4


def _make_sc_gather(epad):
    info = plsc.get_sparse_core_info()
    nc, ns = info.num_cores, info.num_subcores
    nw = nc * ns
    per_w = epad // nw
    n_chunks = per_w // CHUNK
    assert per_w % CHUNK == 0 and n_chunks % 4 == 0 and n_chunks >= 8
    mesh = plsc.VectorSubcoreMesh(core_axis_name="c", subcore_axis_name="s")

    @functools.partial(
        pl.kernel,
        mesh=mesh,
        out_type=jax.ShapeDtypeStruct((epad, D_HALF), jnp.uint32),
        scratch_types=[
            pltpu.VMEM((2, per_w), jnp.int32),
            pltpu.VMEM((NBUF, CHUNK, D_HALF), jnp.uint32),
            pltpu.VMEM((2, CHUNK, D_HALF), jnp.uint32),
            [pltpu.SemaphoreType.DMA] * NBUF,
            [pltpu.SemaphoreType.DMA] * NBUF,
        ],
        compiler_params=pltpu.CompilerParams(use_tc_tiling_on_sc=True,
                                             needs_layout_passes=False),
    )
    def sc_gather(p_hbm, idx_hbm, out_hbm, idx_all, rows_a, rows_b, sg, ss):
        wid = lax.axis_index("s") * nc + lax.axis_index("c")
        wp = wid * per_w
        pltpu.sync_copy(idx_hbm.at[pl.ds(wp, per_w)], idx_all.at[0])
        pltpu.sync_copy(idx_hbm.at[pl.ds(epad + wp, per_w)], idx_all.at[1])

        def fire_gather(g, a, b):
            pltpu.async_copy(p_hbm.at[idx_all.at[0, pl.ds(g * CHUNK, CHUNK)]],
                             rows_a.at[a], sg[a])
            pltpu.async_copy(p_hbm.at[idx_all.at[1, pl.ds(g * CHUNK, CHUNK)]],
                             rows_b.at[b], sg[a])

        def wait_gather(a, b):
            pltpu.make_async_copy(p_hbm.at[idx_all.at[0, pl.ds(0, CHUNK)]],
                                  rows_a.at[a], sg[a]).wait()
            pltpu.make_async_copy(p_hbm.at[idx_all.at[0, pl.ds(0, CHUNK)]],
                                  rows_b.at[b], sg[a]).wait()

        def add_rows(a, b):
            def rowbody(r, _):
                for c in range(D_HALF // 16):
                    x = plsc.bitcast(rows_a[a, r, pl.ds(c * 16, 16)], jnp.bfloat16)
                    y = plsc.bitcast(rows_b[b, r, pl.ds(c * 16, 16)], jnp.bfloat16)
                    rows_a[a, r, pl.ds(c * 16, 16)] = plsc.bitcast(x + y, jnp.uint32)
                return 0
            lax.fori_loop(0, CHUNK, rowbody, 0)

        def fire_store(g, a):
            pltpu.async_copy(rows_a.at[a],
                             out_hbm.at[pl.ds(wp + g * CHUNK, CHUNK)], ss[a])

        def wait_store(a):
            pltpu.make_async_copy(rows_a.at[a],
                                  out_hbm.at[pl.ds(wp, CHUNK)], ss[a]).wait()

        fire_gather(0, 0, 0)
        fire_gather(1, 1, 1)
        for g in (0, 1):
            wait_gather(g, g % 2)
            add_rows(g, g % 2)
            fire_gather(g + 2, g + 2, g % 2)
            fire_store(g, g)

        def quad(h, _):
            for q in range(4):
                g = 2 + h * 4 + q
                a = (2 + q) % 4
                b = q % 2
                wait_store((a + 2) % 4)
                wait_gather(a, b)
                add_rows(a, b)
                fire_gather(g + 2, (a + 2) % 4, b)
                fire_store(g, a)
            return 0

        lax.fori_loop(0, (n_chunks - 4) // 4, quad, 0)

        for g in (n_chunks - 2, n_chunks - 1):
            a = g % 4
            b = g % 2
            wait_store((a + 2) % 4)
            wait_gather(a, b)
            add_rows(a, b)
            fire_store(g, a)
        wait_store((n_chunks - 2) % 4)
        wait_store((n_chunks - 1) % 4)

    return sc_gather


def _bf16_split(x):
    hi = x.astype(jnp.bfloat16)
    lo = (x - hi.astype(jnp.float32)).astype(jnp.bfloat16)
    return hi, lo


def _dot_f32ish(x, w_hi, w_lo):
    x_bf = x.astype(jnp.bfloat16)
    return (jnp.dot(x_bf, w_hi, preferred_element_type=jnp.float32)
            + jnp.dot(x_bf, w_lo, preferred_element_type=jnp.float32))


def _dot_t(w, x, w_dim, x_dim):
    return lax.dot_general(w, x, (((w_dim,), (x_dim,)), ((), ())),
                           preferred_element_type=jnp.float32)


def _mlp_compute(s_ref, w2lh_ref, w2ll_ref, w2hh_ref, w2hl_ref,
                 b2t_ref, w3h_ref, w3l_ref, b3t_ref, out_ref):
    sal, sah = _unpack_bf16_pair(s_ref[...])
    h1l = jnp.maximum(sal, 0.0).astype(jnp.bfloat16)
    h1h = jnp.maximum(sah, 0.0).astype(jnp.bfloat16)
    h2t = (_dot_t(w2lh_ref[...], h1l, 0, 1) + _dot_t(w2ll_ref[...], h1l, 0, 1)
           + _dot_t(w2hh_ref[...], h1h, 0, 1) + _dot_t(w2hl_ref[...], h1h, 0, 1))
    h2t = jnp.maximum(h2t + b2t_ref[...], 0.0).astype(jnp.bfloat16)
    out_ref[...] = (_dot_t(w3h_ref[...], h2t, 0, 0) + _dot_t(w3l_ref[...], h2t, 0, 0)
                    + b3t_ref[...])


def _mlp_body_first(*refs):
    _mlp_compute(*refs)


def _mlp_body_next(*refs):
    _mlp_compute(*refs[:9], refs[10])


def _mlp_chunk(s2, w2s, b2, w3s, b3, n_pairs, n_blocks, col_base, prev):
    wspec = pl.BlockSpec((D_HALF, D_H2), lambda g: (0, 0))
    in_specs = [
        pl.BlockSpec((R_BLK, D_HALF), lambda g: (g, 0)),
        wspec, wspec, wspec, wspec,
        pl.BlockSpec((D_H2, 1), lambda g: (0, 0)),
        pl.BlockSpec((D_H2, D_OUT), lambda g: (0, 0)),
        pl.BlockSpec((D_H2, D_OUT), lambda g: (0, 0)),
        pl.BlockSpec((D_OUT, 1), lambda g: (0, 0)),
    ]
    args = [s2, *w2s, b2, *w3s, b3]
    body = _mlp_body_first
    aliases = {}
    if prev is not None:
        in_specs.append(pl.BlockSpec(memory_space=pl.ANY))
        args.append(prev)
        body = _mlp_body_next
        aliases = {9: 0}
    return pl.pallas_call(
        body,
        grid=(n_blocks,),
        in_specs=in_specs,
        out_specs=pl.BlockSpec((D_OUT, R_BLK), lambda g: (0, g + col_base)),
        out_shape=jax.ShapeDtypeStruct((D_OUT, n_pairs), jnp.float32),
        input_output_aliases=aliases,
    )(*args)


def kernel(x1_o, x2_o, idx, attt, features1, W1, b1, W2, b2, W3, b3):
    n = x1_o.shape[0]
    e = idx.shape[1]
    d1, d2 = x1_o.shape[1], x2_o.shape[1]

    rs = jnp.concatenate((
        jnp.full((d1, 1), 1.0, jnp.float32) * attt[0],
        jnp.full((d2, 1), 1.0, jnp.float32) * attt[1],
        jnp.ones((D_IN - d1 - d2, 1), jnp.float32),
    ), axis=0)
    w_st = jnp.stack((W1[:D_IN], W1[D_IN:]))
    b_st = jnp.stack((b1, jnp.zeros_like(b1)))[:, None, :]

    p = _precompute(x1_o, x2_o, features1, w_st, rs, b_st, n)

    w2s = (
        W2[:D_HALF].astype(jnp.bfloat16),
        (W2[:D_HALF] - W2[:D_HALF].astype(jnp.bfloat16).astype(jnp.float32)).astype(jnp.bfloat16),
        W2[D_HALF:].astype(jnp.bfloat16),
        (W2[D_HALF:] - W2[D_HALF:].astype(jnp.bfloat16).astype(jnp.float32)).astype(jnp.bfloat16),
    )
    w3s = (
        W3.astype(jnp.bfloat16),
        (W3 - W3.astype(jnp.bfloat16).astype(jnp.float32)).astype(jnp.bfloat16),
    )
    b2t, b3t = b2[:, None], b3[:, None]

    GRAN = 16384
    e0 = (e // (2 * GRAN)) * GRAN
    chunks = [(0, e0), (e0, e - e0)] if 0 < e0 < e else [(0, e)]

    prev = None
    for start, sz in chunks:
        epad_k = ((sz + GRAN - 1) // GRAN) * GRAN
        aa_k = jnp.pad(lax.dynamic_slice_in_dim(idx[0], start, sz), (0, epad_k - sz))
        bb_k = jnp.pad(lax.dynamic_slice_in_dim(idx[1], start, sz), (0, epad_k - sz)) + n
        idx_k = jnp.concatenate((aa_k, bb_k))

        s2_k = _make_sc_gather(epad_k)(p, idx_k)

        prev = _mlp_chunk(s2_k, w2s, b2t, w3s, b3t, e,
                          (sz + R_BLK - 1) // R_BLK, start // R_BLK, prev)

    return prev.T
